# baseline (speedup 1.0000x reference)
import functools

import jax
import jax.numpy as jnp
from jax import lax
from jax.experimental import pallas as pl
from jax.experimental.pallas import tpu as pltpu
from jax.experimental.pallas import tpu_sc as plsc

B = 4096
CTX = 20
EMB = 64
LANES = 16

_info = plsc.get_sparse_core_info()
_NC, _NS = _info.num_cores, _info.num_subcores
_NW = _NC * _NS
_BPW = B // _NW


def _sc_gather_mean(xT, emb_table):
    mesh = plsc.VectorSubcoreMesh(core_axis_name="c", subcore_axis_name="s")

    @functools.partial(
        pl.kernel,
        out_type=jax.ShapeDtypeStruct((B, EMB), jnp.float32),
        mesh=mesh,
        compiler_params=pltpu.CompilerParams(use_tc_tiling_on_sc=False),
        scratch_types=[
            pltpu.VMEM((CTX, _BPW), jnp.int32),
            pltpu.VMEM((_BPW, EMB), jnp.float32),
            pltpu.VMEM((_BPW, EMB), jnp.float32),
            pltpu.VMEM((_BPW, EMB), jnp.float32),
            pltpu.SemaphoreType.DMA,
            pltpu.SemaphoreType.DMA,
            pltpu.SemaphoreType.DMA,
        ],
    )
    def sc_kernel(xT_hbm, table_hbm, out_hbm, idx_v, acc_v, buf0, buf1,
                  sem_acc, sem0, sem1):
        wid = lax.axis_index("s") * _NC + lax.axis_index("c")
        base = wid * _BPW
        bufs = (buf0, buf1)
        sems = (sem0, sem1)

        pltpu.sync_copy(xT_hbm.at[:, pl.ds(base, _BPW)], idx_v)

        def add_into_acc(src):
            def row(i, _):
                for k in range(EMB // LANES):
                    sl = (i, pl.ds(k * LANES, LANES))
                    plsc.addupdate(acc_v.at[sl], src[sl])
                return 0
            lax.fori_loop(0, _BPW, row, 0)

        d0 = pltpu.async_copy(table_hbm.at[idx_v.at[0]], acc_v, sem_acc)
        d1 = pltpu.async_copy(table_hbm.at[idx_v.at[1]], bufs[1], sems[1])
        d0.wait()
        prev = d1
        for j in range(2, CTX):
            cur = pltpu.async_copy(table_hbm.at[idx_v.at[j]], bufs[j % 2],
                                   sems[j % 2])
            prev.wait()
            add_into_acc(bufs[(j - 1) % 2])
            prev = cur
        prev.wait()
        add_into_acc(bufs[(CTX - 1) % 2])

        scale = jnp.float32(1.0 / CTX)
        def scale_row(i, _):
            for k in range(EMB // LANES):
                sl = (i, pl.ds(k * LANES, LANES))
                acc_v[sl] = acc_v[sl] * scale
            return 0
        lax.fori_loop(0, _BPW, scale_row, 0)
        pltpu.sync_copy(acc_v, out_hbm.at[pl.ds(base, _BPW)])

    return sc_kernel(xT, emb_table)


_VB = 512
_BB = 4096


def _tc_matmul_bias(mean, W, b2d):
    V = W.shape[0]
    nv = pl.cdiv(V, _VB)

    def mm_body(mean_ref, w_ref, b_ref, out_ref):
        acc = lax.dot_general(mean_ref[...], w_ref[...],
                              (((1,), (1,)), ((), ())),
                              preferred_element_type=jnp.float32)
        out_ref[...] = acc + b_ref[...]

    return pl.pallas_call(
        mm_body,
        grid=(nv,),
        in_specs=[
            pl.BlockSpec((_BB, EMB), lambda j: (0, 0)),
            pl.BlockSpec((_VB, EMB), lambda j: (j, 0)),
            pl.BlockSpec((1, _VB), lambda j: (0, j)),
        ],
        out_specs=pl.BlockSpec((_BB, _VB), lambda j: (0, j)),
        out_shape=jax.ShapeDtypeStruct((B, V), jnp.float32),
        compiler_params=pltpu.CompilerParams(
            dimension_semantics=("arbitrary",),
        ),
    )(mean, W, b2d)


def kernel(x, emb_table, W, b):
    xT = jnp.transpose(x.astype(jnp.int32))
    mean = _sc_gather_mean(xT, emb_table)
    return _tc_matmul_bias(mean, W, b.reshape(1, -1))


# speedup vs baseline: 3.1906x; 3.1906x over previous
import functools

import jax
import jax.numpy as jnp
from jax import lax
from jax.experimental import pallas as pl
from jax.experimental.pallas import tpu as pltpu
from jax.experimental.pallas import tpu_sc as plsc

B = 4096
CTX = 20
EMB = 64
LANES = 16

_info = plsc.get_sparse_core_info()
_NC, _NS = _info.num_cores, _info.num_subcores
_NW = _NC * _NS
_BPW = B // _NW


def _sc_gather_mean(xT, emb_table):
    mesh = plsc.VectorSubcoreMesh(core_axis_name="c", subcore_axis_name="s")

    @functools.partial(
        pl.kernel,
        out_type=jax.ShapeDtypeStruct((B, EMB), jnp.float32),
        mesh=mesh,
        compiler_params=pltpu.CompilerParams(use_tc_tiling_on_sc=False),
        scratch_types=[
            pltpu.VMEM((CTX, _BPW), jnp.int32),
            pltpu.VMEM((_BPW, EMB), jnp.float32),
            pltpu.VMEM((_BPW, EMB), jnp.float32),
            pltpu.VMEM((_BPW, EMB), jnp.float32),
            pltpu.SemaphoreType.DMA,
            pltpu.SemaphoreType.DMA,
            pltpu.SemaphoreType.DMA,
        ],
    )
    def sc_kernel(xT_hbm, table_hbm, out_hbm, idx_v, acc_v, buf0, buf1,
                  sem_acc, sem0, sem1):
        wid = lax.axis_index("s") * _NC + lax.axis_index("c")
        base = wid * _BPW
        bufs = (buf0, buf1)
        sems = (sem0, sem1)

        pltpu.sync_copy(xT_hbm.at[:, pl.ds(base, _BPW)], idx_v)

        def add_into_acc(src):
            def row(i, _):
                for k in range(EMB // LANES):
                    sl = (i, pl.ds(k * LANES, LANES))
                    plsc.addupdate(acc_v.at[sl], src[sl])
                return 0
            lax.fori_loop(0, _BPW, row, 0)

        d0 = pltpu.async_copy(table_hbm.at[idx_v.at[0]], acc_v, sem_acc)
        d1 = pltpu.async_copy(table_hbm.at[idx_v.at[1]], bufs[1], sems[1])
        d0.wait()
        prev = d1
        for j in range(2, CTX):
            cur = pltpu.async_copy(table_hbm.at[idx_v.at[j]], bufs[j % 2],
                                   sems[j % 2])
            prev.wait()
            add_into_acc(bufs[(j - 1) % 2])
            prev = cur
        prev.wait()
        add_into_acc(bufs[(CTX - 1) % 2])

        scale = jnp.float32(1.0 / CTX)
        def scale_row(i, _):
            for k in range(EMB // LANES):
                sl = (i, pl.ds(k * LANES, LANES))
                acc_v[sl] = acc_v[sl] * scale
            return 0
        lax.fori_loop(0, _BPW, scale_row, 0)
        pltpu.sync_copy(acc_v, out_hbm.at[pl.ds(base, _BPW)])

    return sc_kernel(xT, emb_table)


_VB = 512


def _tc_matmul_bias(WT, mean, bcol):
    V = WT.shape[1]
    nv = pl.cdiv(V, _VB)

    def mm_body(wT_ref, mean_ref, b_ref, out_ref):
        acc = lax.dot_general(wT_ref[...], mean_ref[...],
                              (((0,), (1,)), ((), ())),
                              preferred_element_type=jnp.float32)
        out_ref[...] = acc + b_ref[...]

    return pl.pallas_call(
        mm_body,
        grid=(nv,),
        in_specs=[
            pl.BlockSpec((EMB, _VB), lambda j: (0, j)),
            pl.BlockSpec((B, EMB), lambda j: (0, 0)),
            pl.BlockSpec((_VB, 1), lambda j: (j, 0)),
        ],
        out_specs=pl.BlockSpec((_VB, B), lambda j: (j, 0)),
        out_shape=jax.ShapeDtypeStruct((V, B), jnp.float32),
        compiler_params=pltpu.CompilerParams(
            dimension_semantics=("arbitrary",),
        ),
    )(WT, mean, bcol)


def kernel(x, emb_table, W, b):
    xT = jnp.transpose(x.astype(jnp.int32))
    mean = _sc_gather_mean(xT, emb_table)
    scoresT = _tc_matmul_bias(W.T, mean, b.reshape(-1, 1))
    return scoresT.T


# speedup vs baseline: 3.3837x; 1.0605x over previous
import functools

import jax
import jax.numpy as jnp
from jax import lax
from jax.experimental import pallas as pl
from jax.experimental.pallas import tpu as pltpu
from jax.experimental.pallas import tpu_sc as plsc

B = 4096
CTX = 20
EMB = 64
LANES = 16

_info = plsc.get_sparse_core_info()
_NC, _NS = _info.num_cores, _info.num_subcores
_NW = _NC * _NS
_BPW = B // _NW


def _sc_gather_mean(xT, emb_table):
    mesh = plsc.VectorSubcoreMesh(core_axis_name="c", subcore_axis_name="s")

    @functools.partial(
        pl.kernel,
        out_type=jax.ShapeDtypeStruct((B, EMB), jnp.float32),
        mesh=mesh,
        compiler_params=pltpu.CompilerParams(use_tc_tiling_on_sc=False),
        scratch_types=[
            pltpu.VMEM((CTX, _BPW), jnp.int32),
            pltpu.VMEM((_BPW, EMB), jnp.float32),
            pltpu.VMEM((_BPW, EMB), jnp.float32),
            pltpu.VMEM((_BPW, EMB), jnp.float32),
            pltpu.SemaphoreType.DMA,
            pltpu.SemaphoreType.DMA,
            pltpu.SemaphoreType.DMA,
        ],
    )
    def sc_kernel(xT_hbm, table_hbm, out_hbm, idx_v, acc_v, buf0, buf1,
                  sem_acc, sem0, sem1):
        wid = lax.axis_index("s") * _NC + lax.axis_index("c")
        base = wid * _BPW
        bufs = (buf0, buf1)
        sems = (sem0, sem1)

        pltpu.sync_copy(xT_hbm.at[:, pl.ds(base, _BPW)], idx_v)

        def add_into_acc(src):
            def row(i, _):
                for k in range(EMB // LANES):
                    sl = (i, pl.ds(k * LANES, LANES))
                    plsc.addupdate(acc_v.at[sl], src[sl])
                return 0
            lax.fori_loop(0, _BPW, row, 0)

        d0 = pltpu.async_copy(table_hbm.at[idx_v.at[0]], acc_v, sem_acc)
        d1 = pltpu.async_copy(table_hbm.at[idx_v.at[1]], bufs[1], sems[1])
        d0.wait()
        prev = d1
        for j in range(2, CTX):
            cur = pltpu.async_copy(table_hbm.at[idx_v.at[j]], bufs[j % 2],
                                   sems[j % 2])
            prev.wait()
            add_into_acc(bufs[(j - 1) % 2])
            prev = cur
        prev.wait()
        add_into_acc(bufs[(CTX - 1) % 2])

        scale = jnp.float32(1.0 / CTX)
        def scale_row(i, _):
            for k in range(EMB // LANES):
                sl = (i, pl.ds(k * LANES, LANES))
                acc_v[sl] = acc_v[sl] * scale
            return 0
        lax.fori_loop(0, _BPW, scale_row, 0)
        pltpu.sync_copy(acc_v, out_hbm.at[pl.ds(base, _BPW)])

    return sc_kernel(xT, emb_table)


_VB = 512


def _tc_matmul_bias(WT, mean, bcol):
    V = WT.shape[1]
    nv = pl.cdiv(V, _VB)

    def mm_body(wT_ref, mean_ref, b_ref, out_ref):
        acc = lax.dot_general(wT_ref[...], mean_ref[...],
                              (((0,), (1,)), ((), ())),
                              preferred_element_type=jnp.float32)
        out_ref[...] = acc + jnp.transpose(b_ref[...])

    return pl.pallas_call(
        mm_body,
        grid=(nv,),
        in_specs=[
            pl.BlockSpec((EMB, _VB), lambda j: (0, j)),
            pl.BlockSpec((B, EMB), lambda j: (0, 0)),
            pl.BlockSpec((1, _VB), lambda j: (0, j)),
        ],
        out_specs=pl.BlockSpec((_VB, B), lambda j: (j, 0)),
        out_shape=jax.ShapeDtypeStruct((V, B), jnp.float32),
        compiler_params=pltpu.CompilerParams(
            dimension_semantics=("arbitrary",),
        ),
    )(WT, mean, bcol)


def kernel(x, emb_table, W, b):
    xT = jnp.transpose(x.astype(jnp.int32))
    mean = _sc_gather_mean(xT, emb_table)
    scoresT = _tc_matmul_bias(W.T, mean, b.reshape(1, -1))
    return scoresT.T


# speedup vs baseline: 3.3998x; 1.0047x over previous
import functools

import jax
import jax.numpy as jnp
from jax import lax
from jax.experimental import pallas as pl
from jax.experimental.pallas import tpu as pltpu
from jax.experimental.pallas import tpu_sc as plsc

B = 4096
CTX = 20
EMB = 64
LANES = 16

_info = plsc.get_sparse_core_info()
_NC, _NS = _info.num_cores, _info.num_subcores
_NW = _NC * _NS
_BPW = B // _NW


def _sc_gather_mean(xT, emb_table):
    mesh = plsc.VectorSubcoreMesh(core_axis_name="c", subcore_axis_name="s")

    @functools.partial(
        pl.kernel,
        out_type=jax.ShapeDtypeStruct((B, EMB), jnp.float32),
        mesh=mesh,
        compiler_params=pltpu.CompilerParams(use_tc_tiling_on_sc=False),
        scratch_types=[
            pltpu.VMEM((CTX, _BPW), jnp.int32),
            pltpu.VMEM((_BPW, EMB), jnp.float32),
            pltpu.VMEM((_BPW, EMB), jnp.float32),
            pltpu.VMEM((_BPW, EMB), jnp.float32),
            pltpu.SemaphoreType.DMA,
            pltpu.SemaphoreType.DMA,
            pltpu.SemaphoreType.DMA,
        ],
    )
    def sc_kernel(xT_hbm, table_hbm, out_hbm, idx_v, acc_v, buf0, buf1,
                  sem_acc, sem0, sem1):
        wid = lax.axis_index("s") * _NC + lax.axis_index("c")
        base = wid * _BPW
        bufs = (buf0, buf1)
        sems = (sem0, sem1)

        pltpu.sync_copy(xT_hbm.at[:, pl.ds(base, _BPW)], idx_v)

        def add_into_acc(src):
            def row(i, _):
                for k in range(EMB // LANES):
                    sl = (i, pl.ds(k * LANES, LANES))
                    plsc.addupdate(acc_v.at[sl], src[sl])
                return 0
            lax.fori_loop(0, _BPW, row, 0)

        d0 = pltpu.async_copy(table_hbm.at[idx_v.at[0]], acc_v, sem_acc)
        d1 = pltpu.async_copy(table_hbm.at[idx_v.at[1]], bufs[1], sems[1])
        d0.wait()
        prev = d1
        for j in range(2, CTX):
            cur = pltpu.async_copy(table_hbm.at[idx_v.at[j]], bufs[j % 2],
                                   sems[j % 2])
            prev.wait()
            add_into_acc(bufs[(j - 1) % 2])
            prev = cur
        prev.wait()
        add_into_acc(bufs[(CTX - 1) % 2])

        scale = jnp.float32(1.0 / CTX)
        def scale_row(i, _):
            for k in range(EMB // LANES):
                sl = (i, pl.ds(k * LANES, LANES))
                acc_v[sl] = acc_v[sl] * scale
            return 0
        lax.fori_loop(0, _BPW, scale_row, 0)
        pltpu.sync_copy(acc_v, out_hbm.at[pl.ds(base, _BPW)])

    return sc_kernel(xT, emb_table)


_VB = 1024


def _tc_matmul_bias(WT, mean, bcol):
    V = WT.shape[1]
    nv = pl.cdiv(V, _VB)

    def mm_body(wT_ref, mean_ref, b_ref, out_ref):
        acc = lax.dot_general(wT_ref[...], mean_ref[...],
                              (((0,), (1,)), ((), ())),
                              preferred_element_type=jnp.float32)
        out_ref[...] = acc + jnp.transpose(b_ref[...])

    return pl.pallas_call(
        mm_body,
        grid=(nv,),
        in_specs=[
            pl.BlockSpec((EMB, _VB), lambda j: (0, j)),
            pl.BlockSpec((B, EMB), lambda j: (0, 0)),
            pl.BlockSpec((1, _VB), lambda j: (0, j)),
        ],
        out_specs=pl.BlockSpec((_VB, B), lambda j: (j, 0)),
        out_shape=jax.ShapeDtypeStruct((V, B), jnp.float32),
        compiler_params=pltpu.CompilerParams(
            dimension_semantics=("arbitrary",),
        ),
    )(WT, mean, bcol)


def kernel(x, emb_table, W, b):
    xT = jnp.transpose(x.astype(jnp.int32))
    mean = _sc_gather_mean(xT, emb_table)
    scoresT = _tc_matmul_bias(W.T, mean, b.reshape(1, -1))
    return scoresT.T


# speedup vs baseline: 3.4544x; 1.0161x over previous
import functools

import jax
import jax.numpy as jnp
from jax import lax
from jax.experimental import pallas as pl
from jax.experimental.pallas import tpu as pltpu
from jax.experimental.pallas import tpu_sc as plsc

B = 4096
CTX = 20
EMB = 64
LANES = 16

_info = plsc.get_sparse_core_info()
_NC, _NS = _info.num_cores, _info.num_subcores
_NW = _NC * _NS
_BPW = B // _NW


def _sc_gather_mean(xT, emb_table):
    mesh = plsc.VectorSubcoreMesh(core_axis_name="c", subcore_axis_name="s")

    @functools.partial(
        pl.kernel,
        out_type=jax.ShapeDtypeStruct((B, EMB), jnp.float32),
        mesh=mesh,
        compiler_params=pltpu.CompilerParams(use_tc_tiling_on_sc=False),
        scratch_types=[
            pltpu.VMEM((CTX, _BPW), jnp.int32),
            pltpu.VMEM((_BPW, EMB), jnp.float32),
            pltpu.SemaphoreType.DMA,
            pltpu.SemaphoreType.DMA,
        ],
    )
    def sc_kernel(xT_hbm, table_hbm, out_hbm, idx_v, acc_v, sem_acc, sem_add):
        wid = lax.axis_index("s") * _NC + lax.axis_index("c")
        base = wid * _BPW

        pltpu.sync_copy(xT_hbm.at[:, pl.ds(base, _BPW)], idx_v)

        pltpu.async_copy(table_hbm.at[idx_v.at[0]], acc_v, sem_acc).wait()
        adds = [pltpu.async_copy(table_hbm.at[idx_v.at[j]], acc_v, sem_add,
                                 add=True)
                for j in range(1, CTX)]
        for d in adds:
            d.wait()
        pltpu.sync_copy(acc_v, out_hbm.at[pl.ds(base, _BPW)])

    return sc_kernel(xT, emb_table)


_VB = 1024


def _tc_matmul_bias(WT, mean, bcol):
    V = WT.shape[1]
    nv = pl.cdiv(V, _VB)

    def mm_body(wT_ref, mean_ref, b_ref, out_ref):
        acc = lax.dot_general(wT_ref[...] * jnp.float32(1.0 / CTX),
                              mean_ref[...],
                              (((0,), (1,)), ((), ())),
                              preferred_element_type=jnp.float32)
        out_ref[...] = acc + jnp.transpose(b_ref[...])

    return pl.pallas_call(
        mm_body,
        grid=(nv,),
        in_specs=[
            pl.BlockSpec((EMB, _VB), lambda j: (0, j)),
            pl.BlockSpec((B, EMB), lambda j: (0, 0)),
            pl.BlockSpec((1, _VB), lambda j: (0, j)),
        ],
        out_specs=pl.BlockSpec((_VB, B), lambda j: (j, 0)),
        out_shape=jax.ShapeDtypeStruct((V, B), jnp.float32),
        compiler_params=pltpu.CompilerParams(
            dimension_semantics=("arbitrary",),
        ),
    )(WT, mean, bcol)


def kernel(x, emb_table, W, b):
    xT = jnp.transpose(x.astype(jnp.int32))
    mean = _sc_gather_mean(xT, emb_table)
    scoresT = _tc_matmul_bias(W.T, mean, b.reshape(1, -1))
    return scoresT.T
